# baseline (speedup 1.0000x reference)
import functools

import jax
import jax.numpy as jnp
from jax import lax
from jax.experimental import pallas as pl
from jax.experimental.pallas import tpu as pltpu
from jax.experimental.pallas import tpu_sc as plsc

N = 10000
E = 320000
HID = 64

NW = 32
NPW = 320
N_PAD = NW * NPW
EC = 4000
G = 32
NEG = float("-inf")

_MLP_R = 1000


def _mlp_body(two_in, href, aref_or_w1a, *rest):
    if two_in:
        aref = aref_or_w1a
        (w1a, w1b, b1, g, bt, w2, b2, oref) = rest
        hid = jnp.dot(href[...], w1a[...], preferred_element_type=jnp.float32)
        hid = hid + jnp.dot(aref[...], w1b[...],
                            preferred_element_type=jnp.float32)
    else:
        w1a = aref_or_w1a
        (b1, g, bt, w2, b2, oref) = rest
        hid = jnp.dot(href[...], w1a[...], preferred_element_type=jnp.float32)
    hid = hid + b1[...]
    mu = jnp.mean(hid, axis=1, keepdims=True)
    var = jnp.mean((hid - mu) ** 2, axis=1, keepdims=True)
    hid = (hid - mu) * lax.rsqrt(var + 1e-5) * g[...] + bt[...]
    hid = jnp.maximum(hid, 0.0)
    oref[...] = jnp.dot(hid, w2[...],
                        preferred_element_type=jnp.float32) + b2[...]


def _mlp(h_prev, a_prev, W1, b1, g, bt, W2, b2):
    dh = h_prev.shape[1]
    d_out = W2.shape[1]
    grid = (N // _MLP_R,)
    row = lambda i: (i, 0)
    full = lambda i: (0, 0)
    wspec = lambda a: pl.BlockSpec(a.shape, full)
    b1r = b1.reshape(1, HID)
    gr = g.reshape(1, HID)
    btr = bt.reshape(1, HID)
    b2r = b2.reshape(1, d_out)
    if a_prev is not None:
        da = a_prev.shape[1]
        w1a, w1b = W1[:dh], W1[dh:]
        args = (h_prev, a_prev, w1a, w1b, b1r, gr, btr, W2, b2r)
        in_specs = [
            pl.BlockSpec((_MLP_R, dh), row),
            pl.BlockSpec((_MLP_R, da), row),
            wspec(w1a), wspec(w1b), wspec(b1r), wspec(gr), wspec(btr),
            wspec(W2), wspec(b2r),
        ]
    else:
        args = (h_prev, W1, b1r, gr, btr, W2, b2r)
        in_specs = [
            pl.BlockSpec((_MLP_R, dh), row),
            wspec(W1), wspec(b1r), wspec(gr), wspec(btr),
            wspec(W2), wspec(b2r),
        ]
    return pl.pallas_call(
        functools.partial(_mlp_body, a_prev is not None),
        grid=grid,
        in_specs=in_specs,
        out_specs=pl.BlockSpec((_MLP_R, d_out), row),
        out_shape=jax.ShapeDtypeStruct((N, d_out), jnp.float32),
    )(*args)


def _seg_max_body(d, n_sub, h_hbm, src_hbm, dst_hbm, out_hbm,
                  dstb, srcb, midx, mdst, gidx, aggr, stage, sem):
    bin_nodes = NPW // n_sub
    agg_rows = bin_nodes + 1
    nc = 2
    wid = lax.axis_index("s") * nc + lax.axis_index("c")

    for sub in range(n_sub):
        lo = (wid * n_sub + sub) * bin_nodes
        hi = lo + bin_nodes

        def init_row(r, carry):
            for ci in range(d // 16):
                aggr[r, pl.ds(ci * 16, 16)] = jnp.full((16,), NEG,
                                                       jnp.float32)
            return carry
        lax.fori_loop(0, agg_rows, init_row, 0)

        def chunk_body(cidx, carry):
            pltpu.sync_copy(dst_hbm.at[pl.ds(cidx * EC, EC)], dstb)
            pltpu.sync_copy(src_hbm.at[pl.ds(cidx * EC, EC)], srcb)

            def filt(i, cnt):
                dv = dstb[pl.ds(i * 16, 16)]
                sv = srcb[pl.ds(i * 16, 16)]
                m = (dv >= lo) & (dv < hi)
                pos = plsc.cumsum(jnp.where(m, 1, 0).astype(jnp.int32))
                idx = cnt + pos - 1
                plsc.store_scatter(midx, [idx], sv, mask=m)
                plsc.store_scatter(mdst, [idx], dv - lo, mask=m)
                return cnt + pos[15]
            cnt = lax.fori_loop(0, EC // 16, filt, jnp.int32(0))

            zeros = jnp.zeros((16,), jnp.int32)
            dead = jnp.full((16,), bin_nodes, jnp.int32)
            for p in range(G // 16):
                midx[pl.ds(cnt + p * 16, 16)] = zeros
                mdst[pl.ds(cnt + p * 16, 16)] = dead

            ngroups = (cnt + G - 1) // G

            def group_body(gi, carry2):
                for p in range(G // 16):
                    gidx[pl.ds(p * 16, 16)] = midx[pl.ds(gi * G + p * 16, 16)]
                pltpu.async_copy(h_hbm.at[gidx], stage, sem).wait()

                def edge_body(j, carry3):
                    dl = mdst[pl.ds(gi * G + j, 16)][0]
                    for ci in range(d // 16):
                        sl = pl.ds(ci * 16, 16)
                        aggr[dl, sl] = jnp.maximum(aggr[dl, sl],
                                                   stage[j, sl])
                    return carry3
                lax.fori_loop(0, G, edge_body, 0)
                return carry2
            lax.fori_loop(0, ngroups, group_body, 0)
            return carry
        lax.fori_loop(0, E // EC, chunk_body, 0)

        def wb_blk(rb, carry):
            def wb_row(rr, carry2):
                for ci in range(d // 16):
                    sl = pl.ds(ci * 16, 16)
                    v = aggr[rb * G + rr, sl]
                    stage[rr, sl] = jnp.where(v == NEG, 0.0, v)
                return carry2
            lax.fori_loop(0, G, wb_row, 0)
            pltpu.sync_copy(stage, out_hbm.at[pl.ds(lo + rb * G, G)])
            return carry
        lax.fori_loop(0, bin_nodes // G, wb_blk, 0)


def _seg_max(h, src, dst):
    d = h.shape[1]
    n_sub = 1 if d <= 256 else 2
    bin_nodes = NPW // n_sub
    mesh = plsc.VectorSubcoreMesh(core_axis_name="c", subcore_axis_name="s",
                                  num_cores=2, num_subcores=16)
    f = pl.kernel(
        functools.partial(_seg_max_body, d, n_sub),
        out_type=jax.ShapeDtypeStruct((N_PAD, d), jnp.float32),
        mesh=mesh,
        compiler_params=pltpu.CompilerParams(needs_layout_passes=False),
        scratch_types=[
            pltpu.VMEM((EC,), jnp.int32),
            pltpu.VMEM((EC,), jnp.int32),
            pltpu.VMEM((EC + G + 16,), jnp.int32),
            pltpu.VMEM((EC + G + 16,), jnp.int32),
            pltpu.VMEM((G,), jnp.int32),
            pltpu.VMEM((bin_nodes + 1, d), jnp.float32),
            pltpu.VMEM((G, d), jnp.float32),
            pltpu.SemaphoreType.DMA,
        ],
    )
    return f(h, src, dst)


def _final_body(h_ref, a_ref, o_ref):
    i = pl.program_id(0)
    m = jnp.concatenate(
        [jnp.max(h_ref[...], axis=0, keepdims=True),
         jnp.max(a_ref[...], axis=0, keepdims=True)], axis=1)

    @pl.when(i == 0)
    def _():
        o_ref[...] = m

    @pl.when(i > 0)
    def _():
        o_ref[...] = jnp.maximum(o_ref[...], m)

    @pl.when(i == (N // _MLP_R) - 1)
    def _():
        v = o_ref[...]
        o_ref[...] = v * lax.rsqrt(jnp.sum(v * v))


def _finalize(h, a_pad):
    d = h.shape[1]
    row = lambda i: (i, 0)
    out = pl.pallas_call(
        _final_body,
        grid=(N // _MLP_R,),
        in_specs=[pl.BlockSpec((_MLP_R, d), row),
                  pl.BlockSpec((_MLP_R, d), row)],
        out_specs=pl.BlockSpec((1, 2 * d), lambda i: (0, 0)),
        out_shape=jax.ShapeDtypeStruct((1, 2 * d), jnp.float32),
    )(h, a_pad)
    return out.reshape(2 * d)


def kernel(x, edge_index,
           W1_0, b1_0, g_0, bt_0, W2_0, b2_0,
           W1_1, b1_1, g_1, bt_1, W2_1, b2_1,
           W1_2, b1_2, g_2, bt_2, W2_2, b2_2):
    src = edge_index[0]
    dst = edge_index[1]

    h0 = _mlp(x, None, W1_0, b1_0, g_0, bt_0, W2_0, b2_0)
    a0 = _seg_max(h0, src, dst)
    h1 = _mlp(h0, a0, W1_1, b1_1, g_1, bt_1, W2_1, b2_1)
    a1 = _seg_max(h1, src, dst)
    h2 = _mlp(h1, a1, W1_2, b1_2, g_2, bt_2, W2_2, b2_2)
    a2 = _seg_max(h2, src, dst)
    return _finalize(h2, a2)
